# baseline (speedup 1.0000x reference)
import functools

import jax
import jax.numpy as jnp
from jax import lax
from jax.experimental import pallas as pl
from jax.experimental.pallas import tpu as pltpu
from jax.experimental.pallas import tpu_sc as plsc

_B, _S, _D = 4096, 200, 64
_N = _B * _S
_NC, _NS = 2, 16
_NW = _NC * _NS
_BPW = _B // _NW
_L = 16
_NG = 4
_NO = 2
_TP = _BPW + 1

_mesh = plsc.VectorSubcoreMesh(
    core_axis_name="c", subcore_axis_name="s", num_cores=_NC, num_subcores=_NS
)


@functools.partial(
    pl.kernel,
    out_type=jax.ShapeDtypeStruct((_S, _D, _B), jnp.float32),
    mesh=_mesh,
    compiler_params=pltpu.CompilerParams(
        use_tc_tiling_on_sc=False, needs_layout_passes=False
    ),
    scratch_types=[
        pltpu.VMEM((_S, _BPW), jnp.int32),
        pltpu.VMEM((_S * _D,), jnp.float32),
        [pltpu.VMEM((_BPW, _D), jnp.float32) for _ in range(_NG)],
        [pltpu.VMEM((_D, _TP), jnp.float32) for _ in range(_NO)],
        [pltpu.SemaphoreType.DMA for _ in range(_NG)],
        [pltpu.SemaphoreType.DMA for _ in range(_NO)],
    ],
)
def _embed_kernel(xt_hbm, table_hbm, pos_hbm, out_hbm, idx_v, pos_v, gbufs, tbufs, gs, ws):
    wid = lax.axis_index("s") * _NC + lax.axis_index("c")
    bw = wid * _BPW
    pltpu.sync_copy(pos_hbm, pos_v)
    pltpu.sync_copy(xt_hbm.at[:, pl.ds(bw, _BPW)], idx_v)

    def start_gather(s, b):
        pltpu.async_copy(table_hbm.at[idx_v.at[s]], gbufs[b], gs[b])

    def wait_gather(b):
        pltpu.make_async_copy(table_hbm.at[pl.ds(0, _BPW)], gbufs[b], gs[b]).wait()

    def start_write(s, bo):
        pltpu.async_copy(
            tbufs[bo].at[:, pl.ds(0, _BPW)],
            out_hbm.at[s, :, pl.ds(bw, _BPW)],
            ws[bo],
        )

    def wait_write(bo):
        pltpu.make_async_copy(
            tbufs[bo].at[:, pl.ds(0, _BPW)],
            out_hbm.at[0, :, pl.ds(0, _BPW)],
            ws[bo],
        ).wait()

    def trans_add(s, b, bo):
        gb, tb = gbufs[b], tbufs[bo]
        rows = [lax.iota(jnp.int32, _L) + c * _L for c in range(_D // _L)]
        pvs = [pos_v[pl.ds(s * _D + c * _L, _L)] for c in range(_D // _L)]

        def body(j, carry):
            col = jnp.full((_L,), j, jnp.int32)
            for c in range(_D // _L):
                val = gb[j, pl.ds(c * _L, _L)] + pvs[c]
                plsc.store_scatter(tb, [rows[c], col], val)
            return carry

        lax.fori_loop(0, _BPW, body, 0)

    for b in range(_NG):
        start_gather(b, b)

    def outer(i, carry):
        for b in range(_NG):
            s = i * _NG + b
            bo = b % _NO
            wait_gather(b)
            if b >= _NO:
                wait_write(bo)
            else:

                @pl.when(i > 0)
                def _():
                    wait_write(bo)

            trans_add(s, b, bo)
            start_write(s, bo)

            @pl.when(i < _S // _NG - 1)
            def _():
                start_gather(s + _NG, b)

        return carry

    lax.fori_loop(0, _S // _NG, outer, 0)
    for bo in range(_NO):
        wait_write(bo)


def kernel(x, token_table, position_embedding):
    xt = x.astype(jnp.int32).T
    posf = position_embedding.reshape(_S * _D)
    out = _embed_kernel(xt, token_table, posf)
    return out.transpose(2, 0, 1)


# speedup vs baseline: 1.4284x; 1.4284x over previous
import functools

import jax
import jax.numpy as jnp
from jax import lax
from jax.experimental import pallas as pl
from jax.experimental.pallas import tpu as pltpu
from jax.experimental.pallas import tpu_sc as plsc

_B, _S, _D = 4096, 200, 64
_N = _B * _S
_NC, _NS = 2, 16
_NW = _NC * _NS
_BPW = _B // _NW
_L = 16
_NG = 8
_PF = 4
_NO = 2
_TP = _BPW + 1

_mesh = plsc.VectorSubcoreMesh(
    core_axis_name="c", subcore_axis_name="s", num_cores=_NC, num_subcores=_NS
)


@functools.partial(
    pl.kernel,
    out_type=jax.ShapeDtypeStruct((_S, _D, _B), jnp.float32),
    mesh=_mesh,
    compiler_params=pltpu.CompilerParams(
        use_tc_tiling_on_sc=False, needs_layout_passes=False
    ),
    scratch_types=[
        pltpu.VMEM((_S, _BPW), jnp.int32),
        pltpu.VMEM((_S * _D,), jnp.float32),
        [pltpu.VMEM((_BPW, _D), jnp.float32) for _ in range(_NG)],
        [pltpu.VMEM((_D, _TP), jnp.float32) for _ in range(_NO)],
        [pltpu.SemaphoreType.DMA for _ in range(_NG)],
        [pltpu.SemaphoreType.DMA for _ in range(_NO)],
    ],
)
def _embed_kernel(xt_hbm, table_hbm, pos_hbm, out_hbm, idx_v, pos_v, gbufs, tbufs, gs, ws):
    wid = lax.axis_index("s") * _NC + lax.axis_index("c")
    bw = wid * _BPW
    pltpu.sync_copy(pos_hbm, pos_v)
    pltpu.sync_copy(xt_hbm.at[:, pl.ds(bw, _BPW)], idx_v)

    def start_gather(s, b):
        pltpu.async_copy(table_hbm.at[idx_v.at[s]], gbufs[b], gs[b])

    def wait_gather(b):
        pltpu.make_async_copy(table_hbm.at[pl.ds(0, _BPW)], gbufs[b], gs[b]).wait()

    def start_write(s, bo):
        pltpu.async_copy(
            tbufs[bo].at[:, pl.ds(0, _BPW)],
            out_hbm.at[s, :, pl.ds(bw, _BPW)],
            ws[bo],
        )

    def wait_write(bo):
        pltpu.make_async_copy(
            tbufs[bo].at[:, pl.ds(0, _BPW)],
            out_hbm.at[0, :, pl.ds(0, _BPW)],
            ws[bo],
        ).wait()

    def trans_add(s, b, bo):
        gb, tb = gbufs[b], tbufs[bo]
        rows = [lax.iota(jnp.int32, _L) + c * _L for c in range(_D // _L)]
        pvs = [pos_v[pl.ds(s * _D + c * _L, _L)] for c in range(_D // _L)]

        @plsc.parallel_loop(0, _BPW, unroll=4)
        def _(j):
            col = jnp.full((_L,), j, jnp.int32)
            for c in range(_D // _L):
                val = gb[j, pl.ds(c * _L, _L)] + pvs[c]
                plsc.store_scatter(tb, [rows[c], col], val)

    for b in range(_PF):
        start_gather(b, b)

    def outer(i, carry):
        for b8 in range(_NG):
            s = i * _NG + b8
            bo = b8 % _NO
            wait_gather(b8)
            if b8 >= _NO:
                wait_write(bo)
            else:

                @pl.when(i > 0)
                def _():
                    wait_write(bo)

            trans_add(s, b8, bo)
            if b8 >= 1:
                start_write(s - 1, (b8 - 1) % _NO)
            else:

                @pl.when(i > 0)
                def _():
                    start_write(s - 1, (_NO - 1) % _NO)

            @pl.when(s + _PF < _S)
            def _():
                start_gather(s + _PF, (b8 + _PF) % _NG)

        return carry

    lax.fori_loop(0, _S // _NG, outer, 0)
    start_write(_S - 1, (_S - 1) % _NO)
    for bo in range(_NO):
        wait_write(bo)


def kernel(x, token_table, position_embedding):
    xt = x.astype(jnp.int32).T
    posf = position_embedding.reshape(_S * _D)
    out = _embed_kernel(xt, token_table, posf)
    return out.transpose(2, 0, 1)


# speedup vs baseline: 1.8299x; 1.2810x over previous
import functools

import jax
import jax.numpy as jnp
from jax import lax
from jax.experimental import pallas as pl
from jax.experimental.pallas import tpu as pltpu
from jax.experimental.pallas import tpu_sc as plsc

_B, _S, _D = 4096, 200, 64
_NV = 1000000
_N = _B * _S
_NC, _NS = 2, 16
_NW = _NC * _NS
_BPW = _B // _NW
_L = 16
_NG = 8
_PF = 4
_NO = 2
_TP = _BPW + 1

_mesh = plsc.VectorSubcoreMesh(
    core_axis_name="c", subcore_axis_name="s", num_cores=_NC, num_subcores=_NS
)


@functools.partial(
    pl.kernel,
    out_type=jax.ShapeDtypeStruct((_S, _D // 8, _B // 128, 8, 128), jnp.float32),
    name="embed_gather",
    mesh=_mesh,
    compiler_params=pltpu.CompilerParams(
        use_tc_tiling_on_sc=False, needs_layout_passes=False
    ),
    scratch_types=[
        pltpu.VMEM((_S, _BPW), jnp.int32),
        pltpu.VMEM((_S * _D,), jnp.float32),
        [pltpu.VMEM((_BPW, _D), jnp.float32) for _ in range(_NG)],
        [pltpu.VMEM((_D // 8, 8, _TP), jnp.float32) for _ in range(_NO)],
        [pltpu.SemaphoreType.DMA for _ in range(_NG)],
        [pltpu.SemaphoreType.DMA for _ in range(_NO)],
    ],
)
def _embed_kernel(xt_hbm, table_hbm, pos_hbm, out_hbm, idx_v, pos_v, gbufs, tbufs, gs, ws):
    wid = lax.axis_index("s") * _NC + lax.axis_index("c")
    bw = wid * _BPW
    pltpu.sync_copy(pos_hbm, pos_v)
    pltpu.sync_copy(xt_hbm.at[:, pl.ds(bw, _BPW)], idx_v)

    def start_gather(s, b):
        pltpu.async_copy(table_hbm.at[idx_v.at[s]], gbufs[b], gs[b])

    def wait_gather(b):
        pltpu.make_async_copy(table_hbm.at[pl.ds(0, _BPW)], gbufs[b], gs[b]).wait()

    def start_write(s, bo):
        pltpu.async_copy(
            tbufs[bo].at[:, :, pl.ds(0, _BPW)],
            out_hbm.at[s, :, wid, :, :],
            ws[bo],
        )

    def wait_write(bo):
        pltpu.make_async_copy(
            tbufs[bo].at[:, :, pl.ds(0, _BPW)],
            out_hbm.at[0, :, 0, :, :],
            ws[bo],
        ).wait()

    def trans_add(s, b, bo):
        gb, tb = gbufs[b], tbufs[bo]
        drange = [lax.iota(jnp.int32, _L) + c * _L for c in range(_D // _L)]
        dgs = [d // 8 for d in drange]
        drs = [d % 8 for d in drange]
        pvs = [pos_v[pl.ds(s * _D + c * _L, _L)] for c in range(_D // _L)]

        @plsc.parallel_loop(0, _BPW, unroll=4)
        def _(j):
            col = jnp.full((_L,), j, jnp.int32)
            for c in range(_D // _L):
                val = gb[j, pl.ds(c * _L, _L)] + pvs[c]
                plsc.store_scatter(tb, [dgs[c], drs[c], col], val)

    for b in range(_PF):
        start_gather(b, b)

    def outer(i, carry):
        for b8 in range(_NG):
            s = i * _NG + b8
            bo = b8 % _NO
            wait_gather(b8)
            if b8 >= _NO:
                wait_write(bo)
            else:

                @pl.when(i > 0)
                def _():
                    wait_write(bo)

            trans_add(s, b8, bo)
            if b8 >= 1:
                start_write(s - 1, (b8 - 1) % _NO)
            else:

                @pl.when(i > 0)
                def _():
                    start_write(s - 1, (_NO - 1) % _NO)

            @pl.when(s + _PF < _S)
            def _():
                start_gather(s + _PF, (b8 + _PF) % _NG)

        return carry

    lax.fori_loop(0, _S // _NG, outer, 0)
    start_write(_S - 1, (_S - 1) % _NO)
    for bo in range(_NO):
        wait_write(bo)


def kernel(x, token_table, position_embedding):
    xt = x.astype(jnp.int32).T
    posf = position_embedding.reshape(_S * _D)
    out = _embed_kernel(xt, token_table, posf)
    return out.transpose(2, 4, 0, 1, 3).reshape(_B, _S, _D)
